# baseline (speedup 1.0000x reference)
import functools

import jax
import jax.numpy as jnp
from jax import lax
from jax.experimental import pallas as pl
from jax.experimental.pallas import tpu as pltpu
from jax.experimental.pallas import tpu_sc as plsc

_NEG = -1e30
_RB = 512
_JC = 704
_HC = 512


def _mm_nt(a, b):
    return jax.lax.dot_general(a, b, (((1,), (1,)), ((), ())),
                               preferred_element_type=jnp.float32)


def _layer_norm(xb, w, b):
    mu = jnp.mean(xb, axis=-1, keepdims=True)
    var = jnp.mean((xb - mu) ** 2, axis=-1, keepdims=True)
    return (xb - mu) / jnp.sqrt(var + 1e-5) * w + b


def _router_kernel(x_ref, ln1w_ref, ln1b_ref, attnW_ref, ln2w_ref, ln2b_ref,
                   gWp_ref, h_ref, y_ref, ti_ref, rw_ref, *, num_experts):
    xb = x_ref[...]
    hb = xb + _mm_nt(_layer_norm(xb, ln1w_ref[...], ln1b_ref[...]),
                     attnW_ref[...])
    yb = _layer_norm(hb, ln2w_ref[...], ln2b_ref[...])
    logits = _mm_nt(yb, gWp_ref[...])
    tb = logits.shape[0]
    lane = jax.lax.broadcasted_iota(jnp.int32, (tb, 128), 1)
    lm = jnp.where(lane < num_experts, logits, _NEG)
    v1 = jnp.max(lm, axis=1, keepdims=True)
    i1 = jnp.min(jnp.where(lm == v1, lane, 127), axis=1, keepdims=True)
    lm2 = jnp.where(lane == i1, _NEG, lm)
    v2 = jnp.max(lm2, axis=1, keepdims=True)
    i2 = jnp.min(jnp.where(lm2 == v2, lane, 127), axis=1, keepdims=True)
    rw1 = jax.nn.sigmoid(v1 - v2)
    rw2 = 1.0 - rw1
    h_ref[...] = hb
    y_ref[...] = yb.astype(jnp.bfloat16)
    ti_ref[...] = (jnp.where(lane == 0, i1.astype(jnp.float32), 0.0)
                   + jnp.where(lane == 1, i2.astype(jnp.float32), 0.0))
    rw_ref[...] = jnp.where(lane < 64, rw1, rw2)


def _routing_tables_kernel(ti_ref, pos_ref, tbl_ref, cnt_s, run_s,
                           *, num_experts, ntiles, nsteps):
    p = pl.program_id(0)
    i = pl.program_id(1)
    tb = ti_ref.shape[0]
    lane = jax.lax.broadcasted_iota(jnp.int32, (tb, 128), 1).astype(jnp.float32)
    e0 = ti_ref[:, 0:1]
    e1 = ti_ref[:, 1:2]
    m0 = (e0 == lane).astype(jnp.float32)
    m1 = (e1 == lane).astype(jnp.float32)
    mb = m0 + m1
    tile_cnt = jnp.sum(mb, axis=0, keepdims=True)

    @pl.when(jnp.logical_and(p == 0, i == 0))
    def _z0():
        cnt_s[...] = jnp.zeros_like(cnt_s)

    @pl.when(p == 0)
    def _count():
        cnt_s[...] += tile_cnt

    @pl.when(jnp.logical_and(p == 0, i == nsteps - 1))
    def _z1():
        run_s[...] = jnp.zeros_like(run_s)

    @pl.when(p == 1)
    def _emit():
        cnt = cnt_s[...]
        ps = jnp.floor((cnt + (_RB - 1)) / _RB) * _RB
        ri = jax.lax.broadcasted_iota(jnp.int32, (128, 128), 0)
        ci = jax.lax.broadcasted_iota(jnp.int32, (128, 128), 1)
        ut = (ri < ci).astype(jnp.float32)
        pad_off = jax.lax.dot_general(ps, ut, (((1,), (0,)), ((), ())),
                                      preferred_element_type=jnp.float32)
        run = run_s[...]
        bases = pad_off + run
        rr = jax.lax.broadcasted_iota(jnp.int32, (tb, tb), 0)
        cc = jax.lax.broadcasted_iota(jnp.int32, (tb, tb), 1)
        lt = (rr > cc).astype(jnp.float32)
        cum = jax.lax.dot_general(lt, mb, (((1,), (0,)), ((), ())),
                                  preferred_element_type=jnp.float32)
        pos0 = jnp.sum((cum + bases) * m0, axis=1, keepdims=True)
        pos1 = jnp.sum((cum + bases) * m1, axis=1, keepdims=True)
        pos_ref[...] = (jnp.where(lane == 0.0, pos0, 0.0)
                        + jnp.where(lane == 1.0, pos1, 0.0))
        run_s[...] = run + tile_cnt

        @pl.when(i == nsteps - 1)
        def _tables():
            pad_end = pad_off + ps
            lanes1 = jax.lax.broadcasted_iota(jnp.int32, (1, 128), 1
                                              ).astype(jnp.float32)
            tile_start = lanes1 * _RB
            acc = jnp.zeros((1, 128), jnp.float32)
            for e in range(num_experts):
                oh = (jax.lax.broadcasted_iota(jnp.int32, (128, 1), 0) == e
                      ).astype(jnp.float32)
                pe = jax.lax.dot_general(pad_end, oh, (((1,), (0,)), ((), ())),
                                         preferred_element_type=jnp.float32)
                acc += (tile_start >= pe).astype(jnp.float32)
            te = jnp.minimum(acc, float(num_experts - 1))
            ones = jnp.ones((128, 1), jnp.float32)
            total = jax.lax.dot_general(ps, ones, (((1,), (0,)), ((), ())),
                                        preferred_element_type=jnp.float32)
            tv = (tile_start < total).astype(jnp.float32)
            rowi = jax.lax.broadcasted_iota(jnp.int32, (8, 128), 0)
            tbl_ref[...] = (jnp.where(rowi == 0, jnp.broadcast_to(te, (8, 128)), 0.0)
                            + jnp.where(rowi == 1, jnp.broadcast_to(tv, (8, 128)), 0.0))


def _grouped_gemm_kernel(te_ref, tv_ref, ys_ref, wg_ref, wu_ref, wd_ref,
                         z_ref, a_s, *, nj, jc, hc):
    s = pl.program_id(1)
    valid = tv_ref[pl.program_id(0)] > 0

    @pl.when(jnp.logical_and(valid, s < nj))
    def _up():
        ysb = ys_ref[...]
        g = _mm_nt(ysb, wg_ref[0].astype(jnp.bfloat16))
        u = _mm_nt(ysb, wu_ref[0].astype(jnp.bfloat16))
        a_s[:, pl.ds(s * jc, jc)] = (jax.nn.silu(g) * u).astype(jnp.bfloat16)

    @pl.when(jnp.logical_and(valid, s >= nj))
    def _down():
        hk = s - nj
        z_ref[:, pl.ds(hk * hc, hc)] = _mm_nt(
            a_s[...], wd_ref[0].astype(jnp.bfloat16))


def _gather_rows_body(y_hbm, tok_hbm, pos_hbm, ys_hbm, tok_v, pos_v, rows_v,
                     sem1, sem2):
    nc = 2
    wid = lax.axis_index("s") * nc + lax.axis_index("c")
    base = wid * 128
    pltpu.sync_copy(tok_hbm.at[pl.ds(base, 128)], tok_v)
    pltpu.sync_copy(pos_hbm.at[pl.ds(base, 128)], pos_v)
    pltpu.async_copy(y_hbm.at[tok_v], rows_v, sem1).wait()
    pltpu.async_copy(rows_v, ys_hbm.at[pos_v], sem2).wait()


def _run_gather(y_bf, tok, posl, np_rows):
    t, hdim = y_bf.shape
    hw = hdim // 2
    y32 = jax.lax.bitcast_convert_type(y_bf.reshape(t, hw, 2), jnp.uint32)
    y3 = y32.reshape(t, hw // 128, 128)
    mesh = plsc.VectorSubcoreMesh(core_axis_name="c", subcore_axis_name="s")
    ys3 = pl.kernel(
        _gather_rows_body,
        out_type=jax.ShapeDtypeStruct((np_rows, hw // 128, 128), jnp.uint32),
        mesh=mesh,
        scratch_types=[
            pltpu.VMEM((128,), jnp.int32),
            pltpu.VMEM((128,), jnp.int32),
            pltpu.VMEM((128, hw // 128, 128), jnp.uint32),
            pltpu.SemaphoreType.DMA,
            pltpu.SemaphoreType.DMA,
        ],
    )(y3, tok, posl)
    ys = jax.lax.bitcast_convert_type(ys3.reshape(np_rows, hw),
                                      jnp.bfloat16)
    return ys.reshape(np_rows, hdim)


def _combine_body(h_hbm, z_hbm, pos_hbm, rw_hbm, out_hbm,
                  pos_v, z_v, h_v, w_v, o_v, sem):
    nc = 2
    wid = lax.axis_index("s") * nc + lax.axis_index("c")
    for it in range(8):
        tok0 = wid * 64 + it * 8
        pltpu.sync_copy(pos_hbm.at[pl.ds(tok0 * 2, 16)], pos_v)
        pltpu.async_copy(z_hbm.at[pos_v], z_v, sem).wait()
        pltpu.sync_copy(h_hbm.at[pl.ds(tok0, 8)], h_v)
        pltpu.sync_copy(rw_hbm.at[pl.ds(tok0, 8)], w_v)
        for i in range(8):
            w0 = w_v[i, pl.ds(0, 16)]
            w1 = w_v[i, pl.ds(64, 16)]

            def body(c, carry):
                off = c * 16
                z0 = z_v[2 * i, pl.ds(off, 16)]
                z1 = z_v[2 * i + 1, pl.ds(off, 16)]
                hv = h_v[i, pl.ds(off, 16)]
                o_v[i, pl.ds(off, 16)] = hv + w0 * z0 + w1 * z1
                return carry

            lax.fori_loop(0, 64, body, 0)
        pltpu.sync_copy(o_v, out_hbm.at[pl.ds(tok0, 8)])


def _run_combine(h, z, posl, rwv):
    t, hdim = h.shape
    mesh = plsc.VectorSubcoreMesh(core_axis_name="c", subcore_axis_name="s")
    out = pl.kernel(
        _combine_body,
        out_type=jax.ShapeDtypeStruct((t, hdim), jnp.float32),
        mesh=mesh,
        scratch_types=[
            pltpu.VMEM((16,), jnp.int32),
            pltpu.VMEM((16, 1024), jnp.float32),
            pltpu.VMEM((8, 1024), jnp.float32),
            pltpu.VMEM((8, 128), jnp.float32),
            pltpu.VMEM((8, 1024), jnp.float32),
            pltpu.SemaphoreType.DMA,
        ],
    )(h, z, posl, rwv)
    return out


def kernel(x, ln1_w, ln1_b, attn_W, ln2_w, ln2_b, gate_W, Wg, Wu, Wd):
    T, H = x.shape
    E, I, _ = Wg.shape
    TB = min(256, T)
    P = 2 * T
    NT = (P - E) // _RB + E
    NP = NT * _RB
    JC = I // 2
    HC = H // 2
    nj = 2
    nh = 2

    gWp = jnp.pad(gate_W, ((0, 128 - E), (0, 0)))
    ln1w = ln1_w.reshape(1, H)
    ln1b = ln1_b.reshape(1, H)
    ln2w = ln2_w.reshape(1, H)
    ln2b = ln2_b.reshape(1, H)

    h, y_bf, tivf, rwv = pl.pallas_call(
        functools.partial(_router_kernel, num_experts=E),
        grid=(T // TB,),
        in_specs=[
            pl.BlockSpec((TB, H), lambda i: (i, 0)),
            pl.BlockSpec((1, H), lambda i: (0, 0)),
            pl.BlockSpec((1, H), lambda i: (0, 0)),
            pl.BlockSpec((H, H), lambda i: (0, 0)),
            pl.BlockSpec((1, H), lambda i: (0, 0)),
            pl.BlockSpec((1, H), lambda i: (0, 0)),
            pl.BlockSpec((128, H), lambda i: (0, 0)),
        ],
        out_specs=[
            pl.BlockSpec((TB, H), lambda i: (i, 0)),
            pl.BlockSpec((TB, H), lambda i: (i, 0)),
            pl.BlockSpec((TB, 128), lambda i: (i, 0)),
            pl.BlockSpec((TB, 128), lambda i: (i, 0)),
        ],
        out_shape=[
            jax.ShapeDtypeStruct((T, H), jnp.float32),
            jax.ShapeDtypeStruct((T, H), jnp.bfloat16),
            jax.ShapeDtypeStruct((T, 128), jnp.float32),
            jax.ShapeDtypeStruct((T, 128), jnp.float32),
        ],
    )(x, ln1w, ln1b, attn_W, ln2w, ln2b, gWp)

    nsteps = T // TB
    pairpos_f, tbl = pl.pallas_call(
        functools.partial(_routing_tables_kernel, num_experts=E,
                          ntiles=NT, nsteps=nsteps),
        grid=(2, nsteps),
        in_specs=[pl.BlockSpec((TB, 128), lambda p, i: (i, 0))],
        out_specs=[
            pl.BlockSpec((TB, 128), lambda p, i: (i, 0)),
            pl.BlockSpec((8, 128), lambda p, i: (0, 0)),
        ],
        out_shape=[
            jax.ShapeDtypeStruct((T, 128), jnp.float32),
            jax.ShapeDtypeStruct((8, 128), jnp.float32),
        ],
        scratch_shapes=[
            pltpu.VMEM((1, 128), jnp.float32),
            pltpu.VMEM((1, 128), jnp.float32),
        ],
    )(tivf)

    posl = pairpos_f[:, :2].astype(jnp.int32).reshape(P)
    tok = jnp.arange(P, dtype=jnp.int32) // 2
    te = tbl[0, :NT].astype(jnp.int32)
    tv = tbl[1, :NT].astype(jnp.int32)

    ys = _run_gather(y_bf, tok, posl, NP)

    grid_spec = pltpu.PrefetchScalarGridSpec(
        num_scalar_prefetch=2,
        grid=(NT, nj + nh),
        in_specs=[
            pl.BlockSpec((_RB, H), lambda i, s, te_r, tv_r: (i * tv_r[i], 0)),
            pl.BlockSpec((1, JC, H),
                         lambda i, s, te_r, tv_r:
                         (te_r[i], jnp.minimum(s, nj - 1), 0)),
            pl.BlockSpec((1, JC, H),
                         lambda i, s, te_r, tv_r:
                         (te_r[i], jnp.minimum(s, nj - 1), 0)),
            pl.BlockSpec((1, HC, I),
                         lambda i, s, te_r, tv_r:
                         (te_r[i], jnp.maximum(s - nj, 0), 0)),
        ],
        out_specs=pl.BlockSpec((_RB, H), lambda i, s, te_r, tv_r: (i, 0)),
        scratch_shapes=[pltpu.VMEM((_RB, I), jnp.bfloat16)],
    )
    z = pl.pallas_call(
        functools.partial(_grouped_gemm_kernel, nj=nj, jc=JC, hc=HC),
        grid_spec=grid_spec,
        out_shape=jax.ShapeDtypeStruct((NP, H), jnp.float32),
        compiler_params=pltpu.CompilerParams(
            vmem_limit_bytes=63 * 1024 * 1024),
    )(te, tv, ys, Wg, Wu, Wd)

    return _run_combine(h, z, posl, rwv)


# speedup vs baseline: 1.4767x; 1.4767x over previous
import functools

import jax
import jax.numpy as jnp
from jax import lax
from jax.experimental import pallas as pl
from jax.experimental.pallas import tpu as pltpu
from jax.experimental.pallas import tpu_sc as plsc

_NEG = -1e30
_RB = 512
_JC = 704
_HC = 512


def _mm_nt(a, b):
    return jax.lax.dot_general(a, b, (((1,), (1,)), ((), ())),
                               preferred_element_type=jnp.float32)


def _layer_norm(xb, w, b):
    mu = jnp.mean(xb, axis=-1, keepdims=True)
    var = jnp.mean((xb - mu) ** 2, axis=-1, keepdims=True)
    return (xb - mu) / jnp.sqrt(var + 1e-5) * w + b


def _router_kernel(x_ref, ln1w_ref, ln1b_ref, attnW_ref, ln2w_ref, ln2b_ref,
                   gWp_ref, h_ref, y_ref, ti_ref, rw_ref, *, num_experts):
    xb = x_ref[...]
    hb = xb + _mm_nt(_layer_norm(xb, ln1w_ref[...], ln1b_ref[...]),
                     attnW_ref[...])
    yb = _layer_norm(hb, ln2w_ref[...], ln2b_ref[...])
    logits = _mm_nt(yb, gWp_ref[...])
    tb = logits.shape[0]
    lane = jax.lax.broadcasted_iota(jnp.int32, (tb, 128), 1)
    lm = jnp.where(lane < num_experts, logits, _NEG)
    v1 = jnp.max(lm, axis=1, keepdims=True)
    i1 = jnp.min(jnp.where(lm == v1, lane, 127), axis=1, keepdims=True)
    lm2 = jnp.where(lane == i1, _NEG, lm)
    v2 = jnp.max(lm2, axis=1, keepdims=True)
    i2 = jnp.min(jnp.where(lm2 == v2, lane, 127), axis=1, keepdims=True)
    rw1 = jax.nn.sigmoid(v1 - v2)
    rw2 = 1.0 - rw1
    h_ref[...] = hb
    y_ref[...] = yb
    ti_ref[...] = (jnp.where(lane == 0, i1.astype(jnp.float32), 0.0)
                   + jnp.where(lane == 1, i2.astype(jnp.float32), 0.0))
    rw_ref[...] = jnp.where(lane < 64, rw1, rw2)


def _routing_tables_kernel(ti_ref, pos_ref, tbl_ref, cnt_s, run_s,
                           *, num_experts, ntiles, nsteps):
    p = pl.program_id(0)
    i = pl.program_id(1)
    tb = ti_ref.shape[0]
    lane = jax.lax.broadcasted_iota(jnp.int32, (tb, 128), 1).astype(jnp.float32)
    e0 = ti_ref[:, 0:1]
    e1 = ti_ref[:, 1:2]
    m0 = (e0 == lane).astype(jnp.float32)
    m1 = (e1 == lane).astype(jnp.float32)
    mb = m0 + m1
    tile_cnt = jnp.sum(mb, axis=0, keepdims=True)

    @pl.when(jnp.logical_and(p == 0, i == 0))
    def _z0():
        cnt_s[...] = jnp.zeros_like(cnt_s)

    @pl.when(p == 0)
    def _count():
        cnt_s[...] += tile_cnt

    @pl.when(jnp.logical_and(p == 0, i == nsteps - 1))
    def _z1():
        run_s[...] = jnp.zeros_like(run_s)

    @pl.when(p == 1)
    def _emit():
        cnt = cnt_s[...]
        ps = jnp.floor((cnt + (_RB - 1)) / _RB) * _RB
        ri = jax.lax.broadcasted_iota(jnp.int32, (128, 128), 0)
        ci = jax.lax.broadcasted_iota(jnp.int32, (128, 128), 1)
        ut = (ri < ci).astype(jnp.float32)
        pad_off = jax.lax.dot_general(ps, ut, (((1,), (0,)), ((), ())),
                                      preferred_element_type=jnp.float32)
        run = run_s[...]
        bases = pad_off + run
        rr = jax.lax.broadcasted_iota(jnp.int32, (tb, tb), 0)
        cc = jax.lax.broadcasted_iota(jnp.int32, (tb, tb), 1)
        lt = (rr > cc).astype(jnp.float32)
        cum = jax.lax.dot_general(lt, mb, (((1,), (0,)), ((), ())),
                                  preferred_element_type=jnp.float32)
        pos0 = jnp.sum((cum + bases) * m0, axis=1, keepdims=True)
        pos1 = jnp.sum((cum + bases) * m1, axis=1, keepdims=True)
        pos_ref[...] = (jnp.where(lane == 0.0, pos0, 0.0)
                        + jnp.where(lane == 1.0, pos1, 0.0))
        run_s[...] = run + tile_cnt

        @pl.when(i == nsteps - 1)
        def _tables():
            pad_end = pad_off + ps
            lanes1 = jax.lax.broadcasted_iota(jnp.int32, (1, 128), 1
                                              ).astype(jnp.float32)
            tile_start = lanes1 * _RB
            acc = jnp.zeros((1, 128), jnp.float32)
            for e in range(num_experts):
                oh = (jax.lax.broadcasted_iota(jnp.int32, (128, 1), 0) == e
                      ).astype(jnp.float32)
                pe = jax.lax.dot_general(pad_end, oh, (((1,), (0,)), ((), ())),
                                         preferred_element_type=jnp.float32)
                acc += (tile_start >= pe).astype(jnp.float32)
            te = jnp.minimum(acc, float(num_experts - 1))
            ones = jnp.ones((128, 1), jnp.float32)
            total = jax.lax.dot_general(ps, ones, (((1,), (0,)), ((), ())),
                                        preferred_element_type=jnp.float32)
            tv = (tile_start < total).astype(jnp.float32)
            rowi = jax.lax.broadcasted_iota(jnp.int32, (8, 128), 0)
            tbl_ref[...] = (jnp.where(rowi == 0, jnp.broadcast_to(te, (8, 128)), 0.0)
                            + jnp.where(rowi == 1, jnp.broadcast_to(tv, (8, 128)), 0.0))


def _grouped_gemm_kernel(te_ref, tv_ref, ys_ref, wg_ref, wu_ref, wd_ref,
                         z_ref, a_s, *, nj, jc, hc):
    s = pl.program_id(1)
    valid = tv_ref[pl.program_id(0)] > 0

    @pl.when(jnp.logical_and(valid, s < nj))
    def _up():
        ysb = ys_ref[...].astype(jnp.bfloat16)
        g = _mm_nt(ysb, wg_ref[0].astype(jnp.bfloat16))
        u = _mm_nt(ysb, wu_ref[0].astype(jnp.bfloat16))
        a_s[:, pl.ds(s * jc, jc)] = (jax.nn.silu(g) * u).astype(jnp.bfloat16)

    @pl.when(jnp.logical_and(valid, s >= nj))
    def _down():
        hk = s - nj
        z_ref[:, pl.ds(hk * hc, hc)] = _mm_nt(
            a_s[...], wd_ref[0].astype(jnp.bfloat16))


def _gather_rows_body(y_hbm, tok_hbm, pos_hbm, ys_hbm, tok_v, pos_v, rows_v,
                     sem1, sem2):
    nc = 2
    wid = lax.axis_index("s") * nc + lax.axis_index("c")
    for it in range(2):
        base = wid * 128 + it * 64
        pltpu.sync_copy(tok_hbm.at[pl.ds(base, 64)], tok_v)
        pltpu.sync_copy(pos_hbm.at[pl.ds(base, 64)], pos_v)
        pltpu.async_copy(y_hbm.at[tok_v], rows_v, sem1).wait()
        pltpu.async_copy(rows_v, ys_hbm.at[pos_v], sem2).wait()


def _run_gather(y, tok, posl, np_rows):
    t, hdim = y.shape
    y3 = y.reshape(t, hdim // 128, 128)
    mesh = plsc.VectorSubcoreMesh(core_axis_name="c", subcore_axis_name="s")
    ys3 = pl.kernel(
        _gather_rows_body,
        out_type=jax.ShapeDtypeStruct((np_rows, hdim // 128, 128),
                                      jnp.float32),
        mesh=mesh,
        scratch_types=[
            pltpu.VMEM((64,), jnp.int32),
            pltpu.VMEM((64,), jnp.int32),
            pltpu.VMEM((64, hdim // 128, 128), jnp.float32),
            pltpu.SemaphoreType.DMA,
            pltpu.SemaphoreType.DMA,
        ],
    )(y3, tok, posl)
    return ys3.reshape(np_rows, hdim)


def _combine_body(h_hbm, z_hbm, pos_hbm, rw_hbm, out_hbm, *refs):
    nc = 2
    wid = lax.axis_index("s") * nc + lax.axis_index("c")
    bufs = [refs[0:9], refs[9:18]]
    handles = {}
    out_handles = {}

    def start(it):
        pos_v, z_v, h_v, w_v, o_v, sz, sh, sw, _ = bufs[it % 2]
        tok0 = wid * 64 + it * 8
        pltpu.sync_copy(pos_hbm.at[pl.ds(tok0 * 2, 16)], pos_v)
        handles[it] = (
            pltpu.async_copy(z_hbm.at[pos_v], z_v, sz),
            pltpu.async_copy(h_hbm.at[pl.ds(tok0, 8)], h_v, sh),
            pltpu.async_copy(rw_hbm.at[pl.ds(tok0, 8)], w_v, sw),
        )

    start(0)
    for it in range(8):
        if it + 1 < 8:
            start(it + 1)
        for hnd in handles.pop(it):
            hnd.wait()
        if it - 2 in out_handles:
            out_handles.pop(it - 2).wait()
        pos_v, z_v, h_v, w_v, o_v, _, _, _, so = bufs[it % 2]
        tok0 = wid * 64 + it * 8
        for i in range(8):
            w0 = w_v[i, pl.ds(0, 16)]
            w1 = w_v[i, pl.ds(64, 16)]

            def body(c, carry):
                off = c * 16
                z0 = z_v[2 * i, pl.ds(off, 16)]
                z1 = z_v[2 * i + 1, pl.ds(off, 16)]
                hv = h_v[i, pl.ds(off, 16)]
                o_v[i, pl.ds(off, 16)] = hv + w0 * z0 + w1 * z1
                return carry

            lax.fori_loop(0, 64, body, 0)
        out_handles[it] = pltpu.async_copy(o_v, out_hbm.at[pl.ds(tok0, 8)],
                                           so)
    for it in (6, 7):
        out_handles.pop(it).wait()


def _run_combine(h, z, posl, rwv):
    t, hdim = h.shape
    mesh = plsc.VectorSubcoreMesh(core_axis_name="c", subcore_axis_name="s")
    buf = [
        pltpu.VMEM((16,), jnp.int32),
        pltpu.VMEM((16, 1024), jnp.float32),
        pltpu.VMEM((8, 1024), jnp.float32),
        pltpu.VMEM((8, 128), jnp.float32),
        pltpu.VMEM((8, 1024), jnp.float32),
        pltpu.SemaphoreType.DMA,
        pltpu.SemaphoreType.DMA,
        pltpu.SemaphoreType.DMA,
        pltpu.SemaphoreType.DMA,
    ]
    out = pl.kernel(
        _combine_body,
        out_type=jax.ShapeDtypeStruct((t, hdim), jnp.float32),
        mesh=mesh,
        scratch_types=buf + buf,
    )(h, z, posl, rwv)
    return out


def kernel(x, ln1_w, ln1_b, attn_W, ln2_w, ln2_b, gate_W, Wg, Wu, Wd):
    T, H = x.shape
    E, I, _ = Wg.shape
    TB = min(256, T)
    P = 2 * T
    NT = (P - E) // _RB + E
    NP = NT * _RB
    JC = I // 2
    HC = H // 2
    nj = 2
    nh = 2

    gWp = jnp.pad(gate_W, ((0, 128 - E), (0, 0)))
    ln1w = ln1_w.reshape(1, H)
    ln1b = ln1_b.reshape(1, H)
    ln2w = ln2_w.reshape(1, H)
    ln2b = ln2_b.reshape(1, H)

    h, y_bf, tivf, rwv = pl.pallas_call(
        functools.partial(_router_kernel, num_experts=E),
        grid=(T // TB,),
        in_specs=[
            pl.BlockSpec((TB, H), lambda i: (i, 0)),
            pl.BlockSpec((1, H), lambda i: (0, 0)),
            pl.BlockSpec((1, H), lambda i: (0, 0)),
            pl.BlockSpec((H, H), lambda i: (0, 0)),
            pl.BlockSpec((1, H), lambda i: (0, 0)),
            pl.BlockSpec((1, H), lambda i: (0, 0)),
            pl.BlockSpec((128, H), lambda i: (0, 0)),
        ],
        out_specs=[
            pl.BlockSpec((TB, H), lambda i: (i, 0)),
            pl.BlockSpec((TB, H), lambda i: (i, 0)),
            pl.BlockSpec((TB, 128), lambda i: (i, 0)),
            pl.BlockSpec((TB, 128), lambda i: (i, 0)),
        ],
        out_shape=[
            jax.ShapeDtypeStruct((T, H), jnp.float32),
            jax.ShapeDtypeStruct((T, H), jnp.float32),
            jax.ShapeDtypeStruct((T, 128), jnp.float32),
            jax.ShapeDtypeStruct((T, 128), jnp.float32),
        ],
    )(x, ln1w, ln1b, attn_W, ln2w, ln2b, gWp)

    nsteps = T // TB
    pairpos_f, tbl = pl.pallas_call(
        functools.partial(_routing_tables_kernel, num_experts=E,
                          ntiles=NT, nsteps=nsteps),
        grid=(2, nsteps),
        in_specs=[pl.BlockSpec((TB, 128), lambda p, i: (i, 0))],
        out_specs=[
            pl.BlockSpec((TB, 128), lambda p, i: (i, 0)),
            pl.BlockSpec((8, 128), lambda p, i: (0, 0)),
        ],
        out_shape=[
            jax.ShapeDtypeStruct((T, 128), jnp.float32),
            jax.ShapeDtypeStruct((8, 128), jnp.float32),
        ],
        scratch_shapes=[
            pltpu.VMEM((1, 128), jnp.float32),
            pltpu.VMEM((1, 128), jnp.float32),
        ],
    )(tivf)

    posl = pairpos_f[:, :2].astype(jnp.int32).reshape(P)
    tok = jnp.arange(P, dtype=jnp.int32) // 2
    te = tbl[0, :NT].astype(jnp.int32)
    tv = tbl[1, :NT].astype(jnp.int32)

    ys = _run_gather(y_bf, tok, posl, NP)

    grid_spec = pltpu.PrefetchScalarGridSpec(
        num_scalar_prefetch=2,
        grid=(NT, nj + nh),
        in_specs=[
            pl.BlockSpec((_RB, H), lambda i, s, te_r, tv_r: (i * tv_r[i], 0)),
            pl.BlockSpec((1, JC, H),
                         lambda i, s, te_r, tv_r:
                         (te_r[i], jnp.minimum(s, nj - 1), 0)),
            pl.BlockSpec((1, JC, H),
                         lambda i, s, te_r, tv_r:
                         (te_r[i], jnp.minimum(s, nj - 1), 0)),
            pl.BlockSpec((1, HC, I),
                         lambda i, s, te_r, tv_r:
                         (te_r[i], jnp.maximum(s - nj, 0), 0)),
        ],
        out_specs=pl.BlockSpec((_RB, H), lambda i, s, te_r, tv_r: (i, 0)),
        scratch_shapes=[pltpu.VMEM((_RB, I), jnp.bfloat16)],
    )
    z = pl.pallas_call(
        functools.partial(_grouped_gemm_kernel, nj=nj, jc=JC, hc=HC),
        grid_spec=grid_spec,
        out_shape=jax.ShapeDtypeStruct((NP, H), jnp.float32),
        compiler_params=pltpu.CompilerParams(
            vmem_limit_bytes=63 * 1024 * 1024),
    )(te, tv, ys, Wg, Wu, Wd)

    return _run_combine(h, z, posl, rwv)


# speedup vs baseline: 1.6108x; 1.0908x over previous
import functools

import jax
import jax.numpy as jnp
from jax import lax
from jax.experimental import pallas as pl
from jax.experimental.pallas import tpu as pltpu
from jax.experimental.pallas import tpu_sc as plsc

_NEG = -1e30
_RB = 512
_JC = 704
_HC = 512


def _mm_nt(a, b):
    return jax.lax.dot_general(a, b, (((1,), (1,)), ((), ())),
                               preferred_element_type=jnp.float32)


def _layer_norm(xb, w, b):
    mu = jnp.mean(xb, axis=-1, keepdims=True)
    var = jnp.mean((xb - mu) ** 2, axis=-1, keepdims=True)
    return (xb - mu) / jnp.sqrt(var + 1e-5) * w + b


def _router_kernel(x_ref, ln1w_ref, ln1b_ref, attnW_ref, ln2w_ref, ln2b_ref,
                   gWp_ref, h_ref, y_ref, ti_ref, rw_ref, *, num_experts):
    xb = x_ref[...]
    hb = xb + _mm_nt(_layer_norm(xb, ln1w_ref[...], ln1b_ref[...]),
                     attnW_ref[...])
    yb = _layer_norm(hb, ln2w_ref[...], ln2b_ref[...])
    logits = _mm_nt(yb, gWp_ref[...])
    tb = logits.shape[0]
    lane = jax.lax.broadcasted_iota(jnp.int32, (tb, 128), 1)
    lm = jnp.where(lane < num_experts, logits, _NEG)
    v1 = jnp.max(lm, axis=1, keepdims=True)
    i1 = jnp.min(jnp.where(lm == v1, lane, 127), axis=1, keepdims=True)
    lm2 = jnp.where(lane == i1, _NEG, lm)
    v2 = jnp.max(lm2, axis=1, keepdims=True)
    i2 = jnp.min(jnp.where(lm2 == v2, lane, 127), axis=1, keepdims=True)
    rw1 = jax.nn.sigmoid(v1 - v2)
    rw2 = 1.0 - rw1
    h_ref[...] = hb
    y_ref[...] = yb
    ti_ref[...] = (jnp.where(lane == 0, i1.astype(jnp.float32), 0.0)
                   + jnp.where(lane == 1, i2.astype(jnp.float32), 0.0))
    rw_ref[...] = jnp.where(lane < 64, rw1, rw2)


def _routing_tables_kernel(ti_ref, pos_ref, tbl_ref, cnt_s, run_s,
                           *, num_experts, ntiles, nsteps):
    p = pl.program_id(0)
    i = pl.program_id(1)
    tb = ti_ref.shape[0]
    lane = jax.lax.broadcasted_iota(jnp.int32, (tb, 128), 1).astype(jnp.float32)
    e0 = ti_ref[:, 0:1]
    e1 = ti_ref[:, 1:2]
    m0 = (e0 == lane).astype(jnp.float32)
    m1 = (e1 == lane).astype(jnp.float32)
    mb = m0 + m1
    tile_cnt = jnp.sum(mb, axis=0, keepdims=True)

    @pl.when(jnp.logical_and(p == 0, i == 0))
    def _z0():
        cnt_s[...] = jnp.zeros_like(cnt_s)

    @pl.when(p == 0)
    def _count():
        cnt_s[...] += tile_cnt

    @pl.when(jnp.logical_and(p == 0, i == nsteps - 1))
    def _z1():
        run_s[...] = jnp.zeros_like(run_s)

    @pl.when(p == 1)
    def _emit():
        cnt = cnt_s[...]
        ps = jnp.floor((cnt + (_RB - 1)) / _RB) * _RB
        ri = jax.lax.broadcasted_iota(jnp.int32, (128, 128), 0)
        ci = jax.lax.broadcasted_iota(jnp.int32, (128, 128), 1)
        ut = (ri < ci).astype(jnp.float32)
        pad_off = jax.lax.dot_general(ps, ut, (((1,), (0,)), ((), ())),
                                      preferred_element_type=jnp.float32)
        run = run_s[...]
        bases = pad_off + run
        rr = jax.lax.broadcasted_iota(jnp.int32, (tb, tb), 0)
        cc = jax.lax.broadcasted_iota(jnp.int32, (tb, tb), 1)
        lt = (rr > cc).astype(jnp.float32)
        cum = jax.lax.dot_general(lt, mb, (((1,), (0,)), ((), ())),
                                  preferred_element_type=jnp.float32)
        pos0 = jnp.sum((cum + bases) * m0, axis=1, keepdims=True)
        pos1 = jnp.sum((cum + bases) * m1, axis=1, keepdims=True)
        pos_ref[...] = (jnp.where(lane == 0.0, pos0, 0.0)
                        + jnp.where(lane == 1.0, pos1, 0.0))
        run_s[...] = run + tile_cnt

        @pl.when(i == nsteps - 1)
        def _tables():
            pad_end = pad_off + ps
            lanes1 = jax.lax.broadcasted_iota(jnp.int32, (1, 128), 1
                                              ).astype(jnp.float32)
            tile_start = lanes1 * _RB
            acc = jnp.zeros((1, 128), jnp.float32)
            for e in range(num_experts):
                oh = (jax.lax.broadcasted_iota(jnp.int32, (128, 1), 0) == e
                      ).astype(jnp.float32)
                pe = jax.lax.dot_general(pad_end, oh, (((1,), (0,)), ((), ())),
                                         preferred_element_type=jnp.float32)
                acc += (tile_start >= pe).astype(jnp.float32)
            te = jnp.minimum(acc, float(num_experts - 1))
            ones = jnp.ones((128, 1), jnp.float32)
            total = jax.lax.dot_general(ps, ones, (((1,), (0,)), ((), ())),
                                        preferred_element_type=jnp.float32)
            tv = (tile_start < total).astype(jnp.float32)
            rowi = jax.lax.broadcasted_iota(jnp.int32, (8, 128), 0)
            tbl_ref[...] = (jnp.where(rowi == 0, jnp.broadcast_to(te, (8, 128)), 0.0)
                            + jnp.where(rowi == 1, jnp.broadcast_to(tv, (8, 128)), 0.0))


def _grouped_gemm_kernel(te_ref, tv_ref, ys_ref, wg_ref, wu_ref, wd_ref,
                         z_ref, a_s, *, nj, jc, hc):
    s = pl.program_id(1)
    valid = tv_ref[pl.program_id(0)] > 0

    @pl.when(jnp.logical_and(valid, s < nj))
    def _up():
        ysb = ys_ref[...].astype(jnp.bfloat16)
        g = _mm_nt(ysb, wg_ref[0].astype(jnp.bfloat16))
        u = _mm_nt(ysb, wu_ref[0].astype(jnp.bfloat16))
        a_s[:, pl.ds(s * jc, jc)] = (jax.nn.silu(g) * u).astype(jnp.bfloat16)

    @pl.when(jnp.logical_and(valid, s >= nj))
    def _down():
        hk = s - nj
        z_ref[:, pl.ds(hk * hc, hc)] = _mm_nt(
            a_s[...], wd_ref[0].astype(jnp.bfloat16))


def _gather_rows_body(y_hbm, tok_hbm, pos_hbm, ys_hbm, tok_v, pos_v, rows_v,
                     sem1, sem2):
    nc = 2
    wid = lax.axis_index("s") * nc + lax.axis_index("c")
    for it in range(2):
        base = wid * 128 + it * 64
        pltpu.sync_copy(tok_hbm.at[pl.ds(base, 64)], tok_v)
        pltpu.sync_copy(pos_hbm.at[pl.ds(base, 64)], pos_v)
        pltpu.async_copy(y_hbm.at[tok_v], rows_v, sem1).wait()
        pltpu.async_copy(rows_v, ys_hbm.at[pos_v], sem2).wait()


def _run_gather(y, tok, posl, np_rows):
    t, hdim = y.shape
    mesh = plsc.VectorSubcoreMesh(core_axis_name="c", subcore_axis_name="s")
    return pl.kernel(
        _gather_rows_body,
        out_type=jax.ShapeDtypeStruct((np_rows, hdim), jnp.float32),
        mesh=mesh,
        scratch_types=[
            pltpu.VMEM((64,), jnp.int32),
            pltpu.VMEM((64,), jnp.int32),
            pltpu.VMEM((64, hdim), jnp.float32),
            pltpu.SemaphoreType.DMA,
            pltpu.SemaphoreType.DMA,
        ],
    )(y, tok, posl)


def _combine_body(h_hbm, z_hbm, pos_hbm, rw_hbm, out_hbm, *refs):
    nc = 2
    wid = lax.axis_index("s") * nc + lax.axis_index("c")
    bufs = [refs[0:9], refs[9:18]]
    handles = {}
    out_handles = {}

    def start(it):
        pos_v, z_v, h_v, w_v, o_v, sz, sh, sw, _ = bufs[it % 2]
        tok0 = wid * 64 + it * 8
        pltpu.sync_copy(pos_hbm.at[pl.ds(tok0 * 2, 16)], pos_v)
        handles[it] = (
            pltpu.async_copy(z_hbm.at[pos_v], z_v, sz),
            pltpu.async_copy(h_hbm.at[pl.ds(tok0, 8)], h_v, sh),
            pltpu.async_copy(rw_hbm.at[pl.ds(tok0, 8)], w_v, sw),
        )

    start(0)
    for it in range(8):
        if it + 1 < 8:
            start(it + 1)
        for hnd in handles.pop(it):
            hnd.wait()
        if it - 2 in out_handles:
            out_handles.pop(it - 2).wait()
        pos_v, z_v, h_v, w_v, o_v, _, _, _, so = bufs[it % 2]
        tok0 = wid * 64 + it * 8
        for i in range(8):
            w0 = w_v[i, pl.ds(0, 16)]
            w1 = w_v[i, pl.ds(64, 16)]

            def body(c, carry):
                off = c * 16
                z0 = z_v[2 * i, pl.ds(off, 16)]
                z1 = z_v[2 * i + 1, pl.ds(off, 16)]
                hv = h_v[i, pl.ds(off, 16)]
                o_v[i, pl.ds(off, 16)] = hv + w0 * z0 + w1 * z1
                return carry

            lax.fori_loop(0, 64, body, 0)
        out_handles[it] = pltpu.async_copy(o_v, out_hbm.at[pl.ds(tok0, 8)],
                                           so)
    for it in (6, 7):
        out_handles.pop(it).wait()


def _run_combine(h, z, posl, rwv):
    t, hdim = h.shape
    mesh = plsc.VectorSubcoreMesh(core_axis_name="c", subcore_axis_name="s")
    buf = [
        pltpu.VMEM((16,), jnp.int32),
        pltpu.VMEM((16, 1024), jnp.float32),
        pltpu.VMEM((8, 1024), jnp.float32),
        pltpu.VMEM((8, 128), jnp.float32),
        pltpu.VMEM((8, 1024), jnp.float32),
        pltpu.SemaphoreType.DMA,
        pltpu.SemaphoreType.DMA,
        pltpu.SemaphoreType.DMA,
        pltpu.SemaphoreType.DMA,
    ]
    out = pl.kernel(
        _combine_body,
        out_type=jax.ShapeDtypeStruct((t, hdim), jnp.float32),
        mesh=mesh,
        scratch_types=buf + buf,
    )(h, z, posl, rwv)
    return out


def kernel(x, ln1_w, ln1_b, attn_W, ln2_w, ln2_b, gate_W, Wg, Wu, Wd):
    T, H = x.shape
    E, I, _ = Wg.shape
    TB = min(256, T)
    P = 2 * T
    NT = (P - E) // _RB + E
    NP = NT * _RB
    JC = I // 2
    HC = H // 2
    nj = 2
    nh = 2

    gWp = jnp.pad(gate_W, ((0, 128 - E), (0, 0)))
    ln1w = ln1_w.reshape(1, H)
    ln1b = ln1_b.reshape(1, H)
    ln2w = ln2_w.reshape(1, H)
    ln2b = ln2_b.reshape(1, H)

    h, y_bf, tivf, rwv = pl.pallas_call(
        functools.partial(_router_kernel, num_experts=E),
        grid=(T // TB,),
        in_specs=[
            pl.BlockSpec((TB, H), lambda i: (i, 0)),
            pl.BlockSpec((1, H), lambda i: (0, 0)),
            pl.BlockSpec((1, H), lambda i: (0, 0)),
            pl.BlockSpec((H, H), lambda i: (0, 0)),
            pl.BlockSpec((1, H), lambda i: (0, 0)),
            pl.BlockSpec((1, H), lambda i: (0, 0)),
            pl.BlockSpec((128, H), lambda i: (0, 0)),
        ],
        out_specs=[
            pl.BlockSpec((TB, H), lambda i: (i, 0)),
            pl.BlockSpec((TB, H), lambda i: (i, 0)),
            pl.BlockSpec((TB, 128), lambda i: (i, 0)),
            pl.BlockSpec((TB, 128), lambda i: (i, 0)),
        ],
        out_shape=[
            jax.ShapeDtypeStruct((T, H), jnp.float32),
            jax.ShapeDtypeStruct((T, H), jnp.float32),
            jax.ShapeDtypeStruct((T, 128), jnp.float32),
            jax.ShapeDtypeStruct((T, 128), jnp.float32),
        ],
    )(x, ln1w, ln1b, attn_W, ln2w, ln2b, gWp)

    nsteps = T // TB
    pairpos_f, tbl = pl.pallas_call(
        functools.partial(_routing_tables_kernel, num_experts=E,
                          ntiles=NT, nsteps=nsteps),
        grid=(2, nsteps),
        in_specs=[pl.BlockSpec((TB, 128), lambda p, i: (i, 0))],
        out_specs=[
            pl.BlockSpec((TB, 128), lambda p, i: (i, 0)),
            pl.BlockSpec((8, 128), lambda p, i: (0, 0)),
        ],
        out_shape=[
            jax.ShapeDtypeStruct((T, 128), jnp.float32),
            jax.ShapeDtypeStruct((8, 128), jnp.float32),
        ],
        scratch_shapes=[
            pltpu.VMEM((1, 128), jnp.float32),
            pltpu.VMEM((1, 128), jnp.float32),
        ],
    )(tivf)

    posl = pairpos_f[:, :2].astype(jnp.int32).reshape(P)
    tok = jnp.arange(P, dtype=jnp.int32) // 2
    te = tbl[0, :NT].astype(jnp.int32)
    tv = tbl[1, :NT].astype(jnp.int32)

    ys = _run_gather(y_bf, tok, posl, NP)

    grid_spec = pltpu.PrefetchScalarGridSpec(
        num_scalar_prefetch=2,
        grid=(NT, nj + nh),
        in_specs=[
            pl.BlockSpec((_RB, H), lambda i, s, te_r, tv_r: (i * tv_r[i], 0)),
            pl.BlockSpec((1, JC, H),
                         lambda i, s, te_r, tv_r:
                         (te_r[i], jnp.minimum(s, nj - 1), 0)),
            pl.BlockSpec((1, JC, H),
                         lambda i, s, te_r, tv_r:
                         (te_r[i], jnp.minimum(s, nj - 1), 0)),
            pl.BlockSpec((1, HC, I),
                         lambda i, s, te_r, tv_r:
                         (te_r[i], jnp.maximum(s - nj, 0), 0)),
        ],
        out_specs=pl.BlockSpec((_RB, H), lambda i, s, te_r, tv_r: (i, 0)),
        scratch_shapes=[pltpu.VMEM((_RB, I), jnp.bfloat16)],
    )
    z = pl.pallas_call(
        functools.partial(_grouped_gemm_kernel, nj=nj, jc=JC, hc=HC),
        grid_spec=grid_spec,
        out_shape=jax.ShapeDtypeStruct((NP, H), jnp.float32),
        compiler_params=pltpu.CompilerParams(
            vmem_limit_bytes=63 * 1024 * 1024),
    )(te, tv, ys, Wg, Wu, Wd)

    return _run_combine(h, z, posl, rwv)


# speedup vs baseline: 1.6251x; 1.0089x over previous
import functools

import jax
import jax.numpy as jnp
from jax import lax
from jax.experimental import pallas as pl
from jax.experimental.pallas import tpu as pltpu
from jax.experimental.pallas import tpu_sc as plsc

_NEG = -1e30
_RB = 512
_JC = 704
_HC = 512


def _mm_nt(a, b):
    return jax.lax.dot_general(a, b, (((1,), (1,)), ((), ())),
                               preferred_element_type=jnp.float32)


def _layer_norm(xb, w, b):
    mu = jnp.mean(xb, axis=-1, keepdims=True)
    var = jnp.mean((xb - mu) ** 2, axis=-1, keepdims=True)
    return (xb - mu) / jnp.sqrt(var + 1e-5) * w + b


def _router_kernel(x_ref, ln1w_ref, ln1b_ref, attnW_ref, ln2w_ref, ln2b_ref,
                   gWp_ref, h_ref, y_ref, rw_ref, pos_ref, tbl_ref,
                   ti_s, cnt_s, run_s, *, num_experts, nsteps, tb):
    p = pl.program_id(0)
    i = pl.program_id(1)
    lane = jax.lax.broadcasted_iota(jnp.int32, (tb, 128), 1)
    lanef = lane.astype(jnp.float32)

    @pl.when(p == 0)
    def _route():
        xb = x_ref[...]
        hb = xb + _mm_nt(_layer_norm(xb, ln1w_ref[...], ln1b_ref[...]),
                         attnW_ref[...])
        yb = _layer_norm(hb, ln2w_ref[...], ln2b_ref[...])
        logits = _mm_nt(yb, gWp_ref[...])
        lm = jnp.where(lane < num_experts, logits, _NEG)
        v1 = jnp.max(lm, axis=1, keepdims=True)
        i1 = jnp.min(jnp.where(lm == v1, lane, 127), axis=1, keepdims=True)
        lm2 = jnp.where(lane == i1, _NEG, lm)
        v2 = jnp.max(lm2, axis=1, keepdims=True)
        i2 = jnp.min(jnp.where(lm2 == v2, lane, 127), axis=1, keepdims=True)
        rw1 = jax.nn.sigmoid(v1 - v2)
        rw2 = 1.0 - rw1
        h_ref[...] = hb
        y_ref[...] = yb
        tif = (jnp.where(lane == 0, i1.astype(jnp.float32), 0.0)
               + jnp.where(lane == 1, i2.astype(jnp.float32), 0.0))
        ti_s[pl.ds(i * tb, tb), :] = tif
        rw_ref[...] = jnp.where(lane < 64, rw1, rw2)
        m0c = (lane == i1).astype(jnp.float32)
        m1c = (lane == i2).astype(jnp.float32)
        tile_cnt0 = jnp.sum(m0c + m1c, axis=0, keepdims=True)

        @pl.when(i == 0)
        def _z0():
            cnt_s[...] = tile_cnt0

        @pl.when(i != 0)
        def _acc():
            cnt_s[...] += tile_cnt0

        @pl.when(i == nsteps - 1)
        def _z1():
            run_s[...] = jnp.zeros_like(run_s)

    @pl.when(p == 1)
    def _emit():
        tib = ti_s[pl.ds(i * tb, tb), :]
        lane_ = lanef
        e0 = tib[:, 0:1]
        e1 = tib[:, 1:2]
        m0 = (e0 == lane_).astype(jnp.float32)
        m1 = (e1 == lane_).astype(jnp.float32)
        mb = m0 + m1
        tile_cnt = jnp.sum(mb, axis=0, keepdims=True)
        cnt = cnt_s[...]
        ps = jnp.floor((cnt + (_RB - 1)) / _RB) * _RB
        ri = jax.lax.broadcasted_iota(jnp.int32, (128, 128), 0)
        ci = jax.lax.broadcasted_iota(jnp.int32, (128, 128), 1)
        ut = (ri < ci).astype(jnp.float32)
        pad_off = jax.lax.dot_general(ps, ut, (((1,), (0,)), ((), ())),
                                      preferred_element_type=jnp.float32)
        run = run_s[...]
        bases = pad_off + run
        rr = jax.lax.broadcasted_iota(jnp.int32, (tb, tb), 0)
        cc = jax.lax.broadcasted_iota(jnp.int32, (tb, tb), 1)
        lt = (rr > cc).astype(jnp.float32)
        cum = jax.lax.dot_general(lt, mb, (((1,), (0,)), ((), ())),
                                  preferred_element_type=jnp.float32)
        pos0 = jnp.sum((cum + bases) * m0, axis=1, keepdims=True)
        pos1 = jnp.sum((cum + bases) * m1, axis=1, keepdims=True)
        pos_ref[...] = (jnp.where(lane == 0.0, pos0, 0.0)
                        + jnp.where(lane == 1.0, pos1, 0.0))
        run_s[...] = run + tile_cnt

        @pl.when(i == nsteps - 1)
        def _tables():
            pad_end = pad_off + ps
            lanes1 = jax.lax.broadcasted_iota(jnp.int32, (1, 128), 1
                                              ).astype(jnp.float32)
            tile_start = lanes1 * _RB
            acc = jnp.zeros((1, 128), jnp.float32)
            for e in range(num_experts):
                oh = (jax.lax.broadcasted_iota(jnp.int32, (128, 1), 0) == e
                      ).astype(jnp.float32)
                pe = jax.lax.dot_general(pad_end, oh, (((1,), (0,)), ((), ())),
                                         preferred_element_type=jnp.float32)
                acc += (tile_start >= pe).astype(jnp.float32)
            te = jnp.minimum(acc, float(num_experts - 1))
            ones = jnp.ones((128, 1), jnp.float32)
            total = jax.lax.dot_general(ps, ones, (((1,), (0,)), ((), ())),
                                        preferred_element_type=jnp.float32)
            tv = (tile_start < total).astype(jnp.float32)
            rowi = jax.lax.broadcasted_iota(jnp.int32, (8, 128), 0)
            tbl_ref[...] = (jnp.where(rowi == 0, jnp.broadcast_to(te, (8, 128)), 0.0)
                            + jnp.where(rowi == 1, jnp.broadcast_to(tv, (8, 128)), 0.0))


def _grouped_gemm_kernel(te_ref, tv_ref, ys_ref, wg_ref, wu_ref, wd_ref,
                         z_ref, a_s, *, nj, jc, hc):
    s = pl.program_id(1)
    valid = tv_ref[pl.program_id(0)] > 0

    @pl.when(jnp.logical_and(valid, s < nj))
    def _up():
        ysb = ys_ref[...].astype(jnp.bfloat16)
        g = _mm_nt(ysb, wg_ref[0].astype(jnp.bfloat16))
        u = _mm_nt(ysb, wu_ref[0].astype(jnp.bfloat16))
        a_s[:, pl.ds(s * jc, jc)] = (jax.nn.silu(g) * u).astype(jnp.bfloat16)

    @pl.when(jnp.logical_and(valid, s >= nj))
    def _down():
        hk = s - nj
        z_ref[:, pl.ds(hk * hc, hc)] = _mm_nt(
            a_s[...], wd_ref[0].astype(jnp.bfloat16))


def _gather_rows_body(y_hbm, tok_hbm, pos_hbm, ys_hbm, tok_v, pos_v, rows_v,
                     sem1, sem2):
    nc = 2
    wid = lax.axis_index("s") * nc + lax.axis_index("c")
    for it in range(2):
        base = wid * 128 + it * 64
        pltpu.sync_copy(tok_hbm.at[pl.ds(base, 64)], tok_v)
        pltpu.sync_copy(pos_hbm.at[pl.ds(base, 64)], pos_v)
        pltpu.async_copy(y_hbm.at[tok_v], rows_v, sem1).wait()
        pltpu.async_copy(rows_v, ys_hbm.at[pos_v], sem2).wait()


def _run_gather(y, tok, posl, np_rows):
    t, hdim = y.shape
    mesh = plsc.VectorSubcoreMesh(core_axis_name="c", subcore_axis_name="s")
    return pl.kernel(
        _gather_rows_body,
        out_type=jax.ShapeDtypeStruct((np_rows, hdim), jnp.float32),
        mesh=mesh,
        scratch_types=[
            pltpu.VMEM((64,), jnp.int32),
            pltpu.VMEM((64,), jnp.int32),
            pltpu.VMEM((64, hdim), jnp.float32),
            pltpu.SemaphoreType.DMA,
            pltpu.SemaphoreType.DMA,
        ],
    )(y, tok, posl)


def _combine_body(h_hbm, z_hbm, pos_hbm, rw_hbm, out_hbm, *refs):
    nc = 2
    wid = lax.axis_index("s") * nc + lax.axis_index("c")
    bufs = [refs[0:9], refs[9:18]]
    handles = {}
    out_handles = {}

    def start(it):
        pos_v, z_v, h_v, w_v, o_v, sz, sh, sw, _ = bufs[it % 2]
        tok0 = wid * 64 + it * 8
        pltpu.sync_copy(pos_hbm.at[pl.ds(tok0 * 2, 16)], pos_v)
        handles[it] = (
            pltpu.async_copy(z_hbm.at[pos_v], z_v, sz),
            pltpu.async_copy(h_hbm.at[pl.ds(tok0, 8)], h_v, sh),
            pltpu.async_copy(rw_hbm.at[pl.ds(tok0, 8)], w_v, sw),
        )

    start(0)
    for it in range(8):
        if it + 1 < 8:
            start(it + 1)
        for hnd in handles.pop(it):
            hnd.wait()
        if it - 2 in out_handles:
            out_handles.pop(it - 2).wait()
        pos_v, z_v, h_v, w_v, o_v, _, _, _, so = bufs[it % 2]
        tok0 = wid * 64 + it * 8
        for i in range(8):
            w0 = w_v[i, pl.ds(0, 16)]
            w1 = w_v[i, pl.ds(64, 16)]

            def body(c, carry):
                off = c * 16
                z0 = z_v[2 * i, pl.ds(off, 16)]
                z1 = z_v[2 * i + 1, pl.ds(off, 16)]
                hv = h_v[i, pl.ds(off, 16)]
                o_v[i, pl.ds(off, 16)] = hv + w0 * z0 + w1 * z1
                return carry

            lax.fori_loop(0, 64, body, 0)
        out_handles[it] = pltpu.async_copy(o_v, out_hbm.at[pl.ds(tok0, 8)],
                                           so)
    for it in (6, 7):
        out_handles.pop(it).wait()


def _run_combine(h, z, posl, rwv):
    t, hdim = h.shape
    mesh = plsc.VectorSubcoreMesh(core_axis_name="c", subcore_axis_name="s")
    buf = [
        pltpu.VMEM((16,), jnp.int32),
        pltpu.VMEM((16, 1024), jnp.float32),
        pltpu.VMEM((8, 1024), jnp.float32),
        pltpu.VMEM((8, 128), jnp.float32),
        pltpu.VMEM((8, 1024), jnp.float32),
        pltpu.SemaphoreType.DMA,
        pltpu.SemaphoreType.DMA,
        pltpu.SemaphoreType.DMA,
        pltpu.SemaphoreType.DMA,
    ]
    out = pl.kernel(
        _combine_body,
        out_type=jax.ShapeDtypeStruct((t, hdim), jnp.float32),
        mesh=mesh,
        scratch_types=buf + buf,
    )(h, z, posl, rwv)
    return out


def kernel(x, ln1_w, ln1_b, attn_W, ln2_w, ln2_b, gate_W, Wg, Wu, Wd):
    T, H = x.shape
    E, I, _ = Wg.shape
    TB = min(256, T)
    P = 2 * T
    NT = (P - E) // _RB + E
    NP = NT * _RB
    JC = I // 2
    HC = H // 2
    nj = 2
    nh = 2

    gWp = jnp.pad(gate_W, ((0, 128 - E), (0, 0)))
    ln1w = ln1_w.reshape(1, H)
    ln1b = ln1_b.reshape(1, H)
    ln2w = ln2_w.reshape(1, H)
    ln2b = ln2_b.reshape(1, H)

    nsteps = T // TB
    last = nsteps - 1
    h, y_bf, rwv, pairpos_f, tbl = pl.pallas_call(
        functools.partial(_router_kernel, num_experts=E, nsteps=nsteps,
                          tb=TB),
        grid=(2, nsteps),
        in_specs=[
            pl.BlockSpec((TB, H), lambda p, i: (i, 0)),
            pl.BlockSpec((1, H), lambda p, i: (0, 0)),
            pl.BlockSpec((1, H), lambda p, i: (0, 0)),
            pl.BlockSpec((H, H), lambda p, i: (0, 0)),
            pl.BlockSpec((1, H), lambda p, i: (0, 0)),
            pl.BlockSpec((1, H), lambda p, i: (0, 0)),
            pl.BlockSpec((128, H), lambda p, i: (0, 0)),
        ],
        out_specs=[
            pl.BlockSpec((TB, H),
                         lambda p, i: (jnp.where(p == 0, i, last), 0)),
            pl.BlockSpec((TB, H),
                         lambda p, i: (jnp.where(p == 0, i, last), 0)),
            pl.BlockSpec((TB, 128),
                         lambda p, i: (jnp.where(p == 0, i, last), 0)),
            pl.BlockSpec((TB, 128),
                         lambda p, i: (jnp.where(p == 1, i, 0), 0)),
            pl.BlockSpec((8, 128), lambda p, i: (0, 0)),
        ],
        out_shape=[
            jax.ShapeDtypeStruct((T, H), jnp.float32),
            jax.ShapeDtypeStruct((T, H), jnp.float32),
            jax.ShapeDtypeStruct((T, 128), jnp.float32),
            jax.ShapeDtypeStruct((T, 128), jnp.float32),
            jax.ShapeDtypeStruct((8, 128), jnp.float32),
        ],
        scratch_shapes=[
            pltpu.VMEM((T, 128), jnp.float32),
            pltpu.VMEM((1, 128), jnp.float32),
            pltpu.VMEM((1, 128), jnp.float32),
        ],
    )(x, ln1w, ln1b, attn_W, ln2w, ln2b, gWp)

    posl = pairpos_f[:, :2].astype(jnp.int32).reshape(P)
    tok = jnp.arange(P, dtype=jnp.int32) // 2
    te = tbl[0, :NT].astype(jnp.int32)
    tv = tbl[1, :NT].astype(jnp.int32)

    ys = _run_gather(y_bf, tok, posl, NP)

    grid_spec = pltpu.PrefetchScalarGridSpec(
        num_scalar_prefetch=2,
        grid=(NT, nj + nh),
        in_specs=[
            pl.BlockSpec((_RB, H), lambda i, s, te_r, tv_r: (i * tv_r[i], 0)),
            pl.BlockSpec((1, JC, H),
                         lambda i, s, te_r, tv_r:
                         (te_r[i], jnp.minimum(s, nj - 1), 0)),
            pl.BlockSpec((1, JC, H),
                         lambda i, s, te_r, tv_r:
                         (te_r[i], jnp.minimum(s, nj - 1), 0)),
            pl.BlockSpec((1, HC, I),
                         lambda i, s, te_r, tv_r:
                         (te_r[i], jnp.maximum(s - nj, 0), 0)),
        ],
        out_specs=pl.BlockSpec((_RB, H), lambda i, s, te_r, tv_r: (i, 0)),
        scratch_shapes=[pltpu.VMEM((_RB, I), jnp.bfloat16)],
    )
    z = pl.pallas_call(
        functools.partial(_grouped_gemm_kernel, nj=nj, jc=JC, hc=HC),
        grid_spec=grid_spec,
        out_shape=jax.ShapeDtypeStruct((NP, H), jnp.float32),
        compiler_params=pltpu.CompilerParams(
            vmem_limit_bytes=63 * 1024 * 1024),
    )(te, tv, ys, Wg, Wu, Wd)

    return _run_combine(h, z, posl, rwv)
